# baseline (speedup 1.0000x reference)
import jax
import jax.numpy as jnp
from jax.experimental import pallas as pl
from jax.experimental.pallas import tpu as pltpu


def _body(x_ref, emb_ref, w_ref, out_ref, rows_ref):
    b = rows_ref.shape[0]
    for i in range(b):
        rows_ref[i, :] = emb_ref[x_ref[i], :]
    h = rows_ref[...] * jnp.float32(1.5)
    logits = jnp.dot(h, w_ref[...], preferred_element_type=jnp.float32)
    m = jnp.max(logits, axis=1, keepdims=True)
    e = jnp.exp(logits - m)
    out_ref[...] = e / jnp.sum(e, axis=1, keepdims=True)


def kernel(x_init, emb, W, source_p):
    del source_p
    b = x_init.shape[0]
    v, d = emb.shape
    return pl.pallas_call(
        _body,
        in_specs=[
            pl.BlockSpec(memory_space=pltpu.SMEM),
            pl.BlockSpec(memory_space=pltpu.VMEM),
            pl.BlockSpec(memory_space=pltpu.VMEM),
        ],
        out_specs=pl.BlockSpec(memory_space=pltpu.VMEM),
        scratch_shapes=[pltpu.VMEM((b, d), jnp.float32)],
        out_shape=jax.ShapeDtypeStruct((b, v), jnp.float32),
    )(x_init, emb, W)
